# Initial kernel scaffold; baseline (speedup 1.0000x reference)
import jax
import jax.numpy as jnp
from jax.experimental import pallas as pl


def kernel(x, c1_w, c1_b, c2_w, c2_b, c3_w, c3_b, c4_w, c4_b, c5_w, c5_b, f1_w, f1_b, f2_w, f2_b, f3_w, f3_b):
    raise NotImplementedError("write your pallas kernel here")


# speedup vs baseline: 1.0943x; 1.0943x over previous
import functools

import numpy as np

import jax
import jax.numpy as jnp
from jax import lax
from jax.experimental import pallas as pl
from jax.experimental.pallas import tpu as pltpu

_VMEM_BYTES = 56 * 1024 * 1024
_TR = 512

_LRN_ALPHA = 1e-4
_LRN_SIZE = 2
_LRN_K = 1.0


def _cdiv(a, b):
    return (a + b - 1) // b


def _conv_body(xa_ref, xb_ref, w_ref, b_ref, o_ref, a_scr, *, shifts, cin,
               tr, relu, lrn):
    for t, s in enumerate(shifts):
        c0 = t * cin
        if s == 0:
            a_scr[:, c0:c0 + cin] = xa_ref[...]
        else:
            a_scr[0:tr - s, c0:c0 + cin] = xa_ref[s:tr, :]
            a_scr[tr - s:tr, c0:c0 + cin] = xb_ref[0:s, :]

    r = jnp.dot(a_scr[...], w_ref[...], preferred_element_type=jnp.float32)
    r = r + b_ref[...]
    if relu:
        r = jnp.maximum(r, 0.0)
    if lrn:
        xsq = r * r
        prev = pltpu.roll(xsq, 1, axis=1)
        lane = lax.broadcasted_iota(jnp.int32, r.shape, 1)
        prev = jnp.where(lane == 0, 0.0, prev)
        denom = _LRN_K + (_LRN_ALPHA / _LRN_SIZE) * (xsq + prev)
        inv = lax.rsqrt(denom)
        r = r * (inv * jnp.sqrt(inv))
    o_ref[...] = r.astype(o_ref.dtype)


def _conv_flat(x4, w, b, *, kh, kw, relu, lrn):
    n, hp, wp, c = x4.shape
    rows = n * hp * wp
    cout = w.shape[1]
    shifts = tuple(i * wp + j for i in range(kh) for j in range(kw))
    assert w.shape[0] == len(shifts) * c
    assert shifts[-1] < _TR

    ntiles = _cdiv(rows, _TR)
    rp = (ntiles + 1) * _TR
    xf = x4.reshape(rows, c)
    xf = jnp.pad(xf, ((0, rp - rows), (0, 0)))
    bias = b.reshape(1, cout).astype(jnp.float32)

    out = pl.pallas_call(
        functools.partial(_conv_body, shifts=shifts, cin=c, tr=_TR,
                          relu=relu, lrn=lrn),
        out_shape=jax.ShapeDtypeStruct((ntiles * _TR, cout), jnp.bfloat16),
        grid=(ntiles,),
        in_specs=[
            pl.BlockSpec((_TR, c), lambda i: (i, 0)),
            pl.BlockSpec((_TR, c), lambda i: (i + 1, 0)),
            pl.BlockSpec(w.shape, lambda i: (0, 0)),
            pl.BlockSpec((1, cout), lambda i: (0, 0)),
        ],
        out_specs=pl.BlockSpec((_TR, cout), lambda i: (i, 0)),
        scratch_shapes=[pltpu.VMEM((_TR, len(shifts) * c), jnp.bfloat16)],
        compiler_params=pltpu.CompilerParams(
            dimension_semantics=("parallel",),
            vmem_limit_bytes=_VMEM_BYTES),
    )(xf, xf, w, bias)
    return out[:rows].reshape(n, hp, wp, cout)


def _maxpool_valid(x, ho, wo):
    po = (ho - 3) // 2 + 1
    taps = []
    for i in range(3):
        for j in range(3):
            taps.append(x[:, i:i + 2 * po - 1:2, j:j + 2 * po - 1:2, :])
    return functools.reduce(jnp.maximum, taps)


def _conv1_group_input(x_nchw):
    n = x_nchw.shape[0]
    xt = jnp.transpose(x_nchw, (0, 2, 3, 1))
    xt = jnp.pad(xt, ((0, 0), (0, 1), (0, 1), (0, 0)))
    xt = xt.reshape(n, 57, 4, 57, 4, 3)
    xt = jnp.transpose(xt, (0, 1, 3, 2, 4, 5))
    return xt.reshape(n, 57, 57, 48).astype(jnp.bfloat16)


def _conv1_group_rows():
    rows = np.full((432,), 383, dtype=np.int32)
    g = 0
    for ig in range(3):
        for jg in range(3):
            for ir in range(4):
                for jr in range(4):
                    i, j = 4 * ig + ir, 4 * jg + jr
                    if i < 11 and j < 11:
                        for c in range(3):
                            rows[g * 48 + (ir * 4 + jr) * 3 + c] = \
                                (i * 11 + j) * 3 + c
            g += 1
    return rows


_CONV1_ROWS = _conv1_group_rows()


def _fc_body(a_ref, w_ref, b_ref, o_ref, *, relu):
    r = jnp.dot(a_ref[...], w_ref[...], preferred_element_type=jnp.float32)
    r = r + b_ref[...]
    if relu:
        r = jnp.maximum(r, 0.0)
    o_ref[...] = r.astype(o_ref.dtype)


def _fc(a, w, b, *, relu, tn, out_dtype):
    m, k = a.shape
    nn = w.shape[1]
    bias = b.reshape(1, nn).astype(jnp.float32)
    return pl.pallas_call(
        functools.partial(_fc_body, relu=relu),
        out_shape=jax.ShapeDtypeStruct((m, nn), out_dtype),
        grid=(nn // tn,),
        in_specs=[
            pl.BlockSpec((m, k), lambda j: (0, 0)),
            pl.BlockSpec((k, tn), lambda j: (0, j)),
            pl.BlockSpec((1, tn), lambda j: (0, j)),
        ],
        out_specs=pl.BlockSpec((m, tn), lambda j: (0, j)),
        compiler_params=pltpu.CompilerParams(
            dimension_semantics=("parallel",),
            vmem_limit_bytes=_VMEM_BYTES),
    )(a, w, bias)


def kernel(x, c1_w, c1_b, c2_w, c2_b, c3_w, c3_b, c4_w, c4_b, c5_w, c5_b,
           f1_w, f1_b, f2_w, f2_b, f3_w, f3_b):
    n = x.shape[0]

    x1 = _conv1_group_input(x)
    w1 = jnp.take(c1_w, jnp.asarray(_CONV1_ROWS), axis=0)
    y1 = _conv_flat(x1, w1, c1_b, kh=3, kw=3, relu=True, lrn=True)
    p1 = _maxpool_valid(y1, 55, 55)

    x2 = jnp.pad(p1, ((0, 0), (2, 2), (2, 2), (0, 0)))
    y2 = _conv_flat(x2, c2_w, c2_b, kh=5, kw=5, relu=True, lrn=True)
    p2 = _maxpool_valid(y2, 27, 27)

    x3 = jnp.pad(p2, ((0, 0), (1, 1), (1, 1), (0, 0)))
    y3 = _conv_flat(x3, c3_w, c3_b, kh=3, kw=3, relu=True, lrn=False)
    x4 = jnp.pad(y3[:, :13, :13, :], ((0, 0), (1, 1), (1, 1), (0, 0)))
    y4 = _conv_flat(x4, c4_w, c4_b, kh=3, kw=3, relu=True, lrn=False)
    x5 = jnp.pad(y4[:, :13, :13, :], ((0, 0), (1, 1), (1, 1), (0, 0)))
    y5 = _conv_flat(x5, c5_w, c5_b, kh=3, kw=3, relu=True, lrn=False)
    p5 = _maxpool_valid(y5, 13, 13)

    f_in = jnp.transpose(p5, (0, 3, 1, 2)).reshape(n, 256 * 6 * 6)
    h1 = _fc(f_in, f1_w, f1_b, relu=True, tn=512, out_dtype=jnp.bfloat16)
    h2 = _fc(h1, f2_w, f2_b, relu=True, tn=512, out_dtype=jnp.bfloat16)
    h3 = _fc(h2, f3_w, f3_b, relu=False, tn=512, out_dtype=jnp.float32)
    return h3[:, :1000]


# speedup vs baseline: 1.3969x; 1.2765x over previous
import functools

import numpy as np

import jax
import jax.numpy as jnp
from jax import lax
from jax.experimental import pallas as pl
from jax.experimental.pallas import tpu as pltpu

_VMEM_BYTES = 56 * 1024 * 1024
_TR = 512

_LRN_ALPHA = 1e-4
_LRN_SIZE = 2
_LRN_K = 1.0


def _cdiv(a, b):
    return (a + b - 1) // b


def _conv_body(xa_ref, xb_ref, w_ref, b_ref, o_ref, a_scr, *, shifts, cin,
               tr, relu, lrn):
    for t, s in enumerate(shifts):
        c0 = t * cin
        if s == 0:
            a_scr[:, c0:c0 + cin] = xa_ref[...]
        else:
            a_scr[0:tr - s, c0:c0 + cin] = xa_ref[s:tr, :]
            a_scr[tr - s:tr, c0:c0 + cin] = xb_ref[0:s, :]

    r = jnp.dot(a_scr[...], w_ref[...], preferred_element_type=jnp.float32)
    r = r + b_ref[...]
    if relu:
        r = jnp.maximum(r, 0.0)
    if lrn:
        xsq = r * r
        prev = pltpu.roll(xsq, 1, axis=1)
        lane = lax.broadcasted_iota(jnp.int32, r.shape, 1)
        prev = jnp.where(lane == 0, 0.0, prev)
        denom = _LRN_K + (_LRN_ALPHA / _LRN_SIZE) * (xsq + prev)
        inv = lax.rsqrt(denom)
        r = r * (inv * jnp.sqrt(inv))
    o_ref[...] = r.astype(o_ref.dtype)


def _conv_flat(x4, w, b, *, kh, kw, relu, lrn):
    n, hp, wp, c = x4.shape
    rows = n * hp * wp
    cout = w.shape[1]
    shifts = tuple(i * wp + j for i in range(kh) for j in range(kw))
    assert w.shape[0] == len(shifts) * c
    assert shifts[-1] < _TR

    ntiles = _cdiv(rows, _TR)
    rp = (ntiles + 1) * _TR
    xf = x4.reshape(rows, c)
    xf = jnp.pad(xf, ((0, rp - rows), (0, 0)))
    bias = b.reshape(1, cout).astype(jnp.float32)

    out = pl.pallas_call(
        functools.partial(_conv_body, shifts=shifts, cin=c, tr=_TR,
                          relu=relu, lrn=lrn),
        out_shape=jax.ShapeDtypeStruct((ntiles * _TR, cout), jnp.bfloat16),
        grid=(ntiles,),
        in_specs=[
            pl.BlockSpec((_TR, c), lambda i: (i, 0)),
            pl.BlockSpec((_TR, c), lambda i: (i + 1, 0)),
            pl.BlockSpec(w.shape, lambda i: (0, 0)),
            pl.BlockSpec((1, cout), lambda i: (0, 0)),
        ],
        out_specs=pl.BlockSpec((_TR, cout), lambda i: (i, 0)),
        scratch_shapes=[pltpu.VMEM((_TR, len(shifts) * c), jnp.bfloat16)],
        compiler_params=pltpu.CompilerParams(
            dimension_semantics=("parallel",),
            vmem_limit_bytes=_VMEM_BYTES),
    )(xf, xf, w, bias)
    return out[:rows].reshape(n, hp, wp, cout)


def _maxpool_valid(x, ho, wo):
    po = (ho - 3) // 2 + 1
    taps = []
    for i in range(3):
        for j in range(3):
            taps.append(x[:, i:i + 2 * po - 1:2, j:j + 2 * po - 1:2, :])
    return functools.reduce(jnp.maximum, taps)


def _conv1_group_input(x_nchw):
    xt = jnp.transpose(x_nchw, (0, 2, 3, 1))
    xt = jnp.pad(xt, ((0, 0), (0, 1), (0, 1), (0, 0)))
    pieces = [xt[:, ir::4, jr::4, :]
              for ir in range(4) for jr in range(4)]
    return jnp.concatenate(pieces, axis=-1).astype(jnp.bfloat16)


def _conv1_group_rows():
    rows = np.full((432,), 383, dtype=np.int32)
    g = 0
    for ig in range(3):
        for jg in range(3):
            for ir in range(4):
                for jr in range(4):
                    i, j = 4 * ig + ir, 4 * jg + jr
                    if i < 11 and j < 11:
                        for c in range(3):
                            rows[g * 48 + (ir * 4 + jr) * 3 + c] = \
                                (i * 11 + j) * 3 + c
            g += 1
    return rows


_CONV1_ROWS = _conv1_group_rows()


def _fc_body(a_ref, w_ref, b_ref, o_ref, *, relu):
    r = jnp.dot(a_ref[...], w_ref[...], preferred_element_type=jnp.float32)
    r = r + b_ref[...]
    if relu:
        r = jnp.maximum(r, 0.0)
    o_ref[...] = r.astype(o_ref.dtype)


def _fc(a, w, b, *, relu, tn, out_dtype):
    m, k = a.shape
    nn = w.shape[1]
    bias = b.reshape(1, nn).astype(jnp.float32)
    return pl.pallas_call(
        functools.partial(_fc_body, relu=relu),
        out_shape=jax.ShapeDtypeStruct((m, nn), out_dtype),
        grid=(nn // tn,),
        in_specs=[
            pl.BlockSpec((m, k), lambda j: (0, 0)),
            pl.BlockSpec((k, tn), lambda j: (0, j)),
            pl.BlockSpec((1, tn), lambda j: (0, j)),
        ],
        out_specs=pl.BlockSpec((m, tn), lambda j: (0, j)),
        compiler_params=pltpu.CompilerParams(
            dimension_semantics=("parallel",),
            vmem_limit_bytes=_VMEM_BYTES),
    )(a, w, bias)


def kernel(x, c1_w, c1_b, c2_w, c2_b, c3_w, c3_b, c4_w, c4_b, c5_w, c5_b,
           f1_w, f1_b, f2_w, f2_b, f3_w, f3_b):
    n = x.shape[0]

    x1 = _conv1_group_input(x)
    w1 = jnp.take(c1_w, jnp.asarray(_CONV1_ROWS), axis=0)
    y1 = _conv_flat(x1, w1, c1_b, kh=3, kw=3, relu=True, lrn=True)
    p1 = _maxpool_valid(y1, 55, 55)

    x2 = jnp.pad(p1, ((0, 0), (2, 2), (2, 2), (0, 0)))
    y2 = _conv_flat(x2, c2_w, c2_b, kh=5, kw=5, relu=True, lrn=True)
    p2 = _maxpool_valid(y2, 27, 27)

    x3 = jnp.pad(p2, ((0, 0), (1, 1), (1, 1), (0, 0)))
    y3 = _conv_flat(x3, c3_w, c3_b, kh=3, kw=3, relu=True, lrn=False)
    x4 = jnp.pad(y3[:, :13, :13, :], ((0, 0), (1, 1), (1, 1), (0, 0)))
    y4 = _conv_flat(x4, c4_w, c4_b, kh=3, kw=3, relu=True, lrn=False)
    x5 = jnp.pad(y4[:, :13, :13, :], ((0, 0), (1, 1), (1, 1), (0, 0)))
    y5 = _conv_flat(x5, c5_w, c5_b, kh=3, kw=3, relu=True, lrn=False)
    p5 = _maxpool_valid(y5, 13, 13)

    f_in = jnp.transpose(p5, (0, 3, 1, 2)).reshape(n, 256 * 6 * 6)
    h1 = _fc(f_in, f1_w, f1_b, relu=True, tn=512, out_dtype=jnp.bfloat16)
    h2 = _fc(h1, f2_w, f2_b, relu=True, tn=512, out_dtype=jnp.bfloat16)
    h3 = _fc(h2, f3_w, f3_b, relu=False, tn=512, out_dtype=jnp.float32)
    return h3[:, :1000]


# speedup vs baseline: 2.3974x; 1.7162x over previous
import functools

import numpy as np

import jax
import jax.numpy as jnp
from jax import lax
from jax.experimental import pallas as pl
from jax.experimental.pallas import tpu as pltpu

_VMEM_BYTES = 56 * 1024 * 1024
_TR = 512

_LRN_ALPHA = 1e-4
_LRN_SIZE = 2
_LRN_K = 1.0


def _cdiv(a, b):
    return (a + b - 1) // b


def _conv_body(xa_ref, xb_ref, w_ref, b_ref, o_ref, a_scr, *, shifts, cin,
               tr, relu, lrn):
    for t, s in enumerate(shifts):
        c0 = t * cin
        if s == 0:
            a_scr[:, c0:c0 + cin] = xa_ref[...]
        else:
            a_scr[0:tr - s, c0:c0 + cin] = xa_ref[s:tr, :]
            a_scr[tr - s:tr, c0:c0 + cin] = xb_ref[0:s, :]

    r = jnp.dot(a_scr[...], w_ref[...], preferred_element_type=jnp.float32)
    r = r + b_ref[...]
    if relu:
        r = jnp.maximum(r, 0.0)
    if lrn:
        xsq = r * r
        prev = pltpu.roll(xsq, 1, axis=1)
        lane = lax.broadcasted_iota(jnp.int32, r.shape, 1)
        prev = jnp.where(lane == 0, 0.0, prev)
        denom = _LRN_K + (_LRN_ALPHA / _LRN_SIZE) * (xsq + prev)
        inv = lax.rsqrt(denom)
        r = r * (inv * jnp.sqrt(inv))
    o_ref[...] = r.astype(o_ref.dtype)


def _conv_flat(x4, w, b, *, kh, kw, relu, lrn):
    n, hp, wp, c = x4.shape
    rows = n * hp * wp
    cout = w.shape[1]
    shifts = tuple(i * wp + j for i in range(kh) for j in range(kw))
    assert w.shape[0] == len(shifts) * c
    assert shifts[-1] < _TR

    ntiles = _cdiv(rows, _TR)
    rp = (ntiles + 1) * _TR
    xf = x4.reshape(rows, c)
    xf = jnp.pad(xf, ((0, rp - rows), (0, 0)))
    bias = b.reshape(1, cout).astype(jnp.float32)

    out = pl.pallas_call(
        functools.partial(_conv_body, shifts=shifts, cin=c, tr=_TR,
                          relu=relu, lrn=lrn),
        out_shape=jax.ShapeDtypeStruct((ntiles * _TR, cout), jnp.bfloat16),
        grid=(ntiles,),
        in_specs=[
            pl.BlockSpec((_TR, c), lambda i: (i, 0)),
            pl.BlockSpec((_TR, c), lambda i: (i + 1, 0)),
            pl.BlockSpec(w.shape, lambda i: (0, 0)),
            pl.BlockSpec((1, cout), lambda i: (0, 0)),
        ],
        out_specs=pl.BlockSpec((_TR, cout), lambda i: (i, 0)),
        scratch_shapes=[pltpu.VMEM((_TR, len(shifts) * c), jnp.bfloat16)],
        compiler_params=pltpu.CompilerParams(
            dimension_semantics=("parallel",),
            vmem_limit_bytes=_VMEM_BYTES),
    )(xf, xf, w, bias)
    return out[:rows].reshape(n, hp, wp, cout)


def _maxpool_valid(x, ho, wo):
    po = (ho - 3) // 2 + 1
    ee = x[:, 0::2, 0::2, :]
    eo = x[:, 0::2, 1::2, :]
    oe = x[:, 1::2, 0::2, :]
    oo = x[:, 1::2, 1::2, :]

    def s(a, di, dj):
        return a[:, di:di + po, dj:dj + po, :]

    taps = [s(ee, 0, 0), s(eo, 0, 0), s(ee, 0, 1),
            s(oe, 0, 0), s(oo, 0, 0), s(oe, 0, 1),
            s(ee, 1, 0), s(eo, 1, 0), s(ee, 1, 1)]
    return functools.reduce(jnp.maximum, taps)


def _conv1_group_input(x_nchw):
    xt = jnp.transpose(x_nchw, (0, 2, 3, 1))
    xt = jnp.pad(xt, ((0, 0), (0, 1), (0, 1), (0, 0)))
    pieces = [xt[:, ir::4, jr::4, :]
              for ir in range(4) for jr in range(4)]
    return jnp.concatenate(pieces, axis=-1).astype(jnp.bfloat16)


def _conv1_group_rows():
    rows = np.full((432,), 383, dtype=np.int32)
    g = 0
    for ig in range(3):
        for jg in range(3):
            for ir in range(4):
                for jr in range(4):
                    i, j = 4 * ig + ir, 4 * jg + jr
                    if i < 11 and j < 11:
                        for c in range(3):
                            rows[g * 48 + (ir * 4 + jr) * 3 + c] = \
                                (i * 11 + j) * 3 + c
            g += 1
    return rows


_CONV1_ROWS = _conv1_group_rows()


def _fc_body(a_ref, w_ref, b_ref, o_ref, *, relu):
    r = jnp.dot(a_ref[...], w_ref[...], preferred_element_type=jnp.float32)
    r = r + b_ref[...]
    if relu:
        r = jnp.maximum(r, 0.0)
    o_ref[...] = r.astype(o_ref.dtype)


def _fc(a, w, b, *, relu, tn, out_dtype):
    m, k = a.shape
    nn = w.shape[1]
    bias = b.reshape(1, nn).astype(jnp.float32)
    return pl.pallas_call(
        functools.partial(_fc_body, relu=relu),
        out_shape=jax.ShapeDtypeStruct((m, nn), out_dtype),
        grid=(nn // tn,),
        in_specs=[
            pl.BlockSpec((m, k), lambda j: (0, 0)),
            pl.BlockSpec((k, tn), lambda j: (0, j)),
            pl.BlockSpec((1, tn), lambda j: (0, j)),
        ],
        out_specs=pl.BlockSpec((m, tn), lambda j: (0, j)),
        compiler_params=pltpu.CompilerParams(
            dimension_semantics=("parallel",),
            vmem_limit_bytes=_VMEM_BYTES),
    )(a, w, bias)


def kernel(x, c1_w, c1_b, c2_w, c2_b, c3_w, c3_b, c4_w, c4_b, c5_w, c5_b,
           f1_w, f1_b, f2_w, f2_b, f3_w, f3_b):
    n = x.shape[0]

    x1 = _conv1_group_input(x)
    w1 = jnp.take(c1_w, jnp.asarray(_CONV1_ROWS), axis=0)
    y1 = _conv_flat(x1, w1, c1_b, kh=3, kw=3, relu=True, lrn=True)
    p1 = _maxpool_valid(y1, 55, 55)

    x2 = jnp.pad(p1, ((0, 0), (2, 2), (2, 2), (0, 0)))
    y2 = _conv_flat(x2, c2_w, c2_b, kh=5, kw=5, relu=True, lrn=True)
    p2 = _maxpool_valid(y2, 27, 27)

    x3 = jnp.pad(p2, ((0, 0), (1, 1), (1, 1), (0, 0)))
    y3 = _conv_flat(x3, c3_w, c3_b, kh=3, kw=3, relu=True, lrn=False)
    x4 = jnp.pad(y3[:, :13, :13, :], ((0, 0), (1, 1), (1, 1), (0, 0)))
    y4 = _conv_flat(x4, c4_w, c4_b, kh=3, kw=3, relu=True, lrn=False)
    x5 = jnp.pad(y4[:, :13, :13, :], ((0, 0), (1, 1), (1, 1), (0, 0)))
    y5 = _conv_flat(x5, c5_w, c5_b, kh=3, kw=3, relu=True, lrn=False)
    p5 = _maxpool_valid(y5, 13, 13)

    f_in = jnp.transpose(p5, (0, 3, 1, 2)).reshape(n, 256 * 6 * 6)
    h1 = _fc(f_in, f1_w, f1_b, relu=True, tn=512, out_dtype=jnp.bfloat16)
    h2 = _fc(h1, f2_w, f2_b, relu=True, tn=512, out_dtype=jnp.bfloat16)
    h3 = _fc(h2, f3_w, f3_b, relu=False, tn=512, out_dtype=jnp.float32)
    return h3[:, :1000]


# speedup vs baseline: 8.0251x; 3.3475x over previous
import functools

import numpy as np

import jax
import jax.numpy as jnp
from jax import lax
from jax.experimental import pallas as pl
from jax.experimental.pallas import tpu as pltpu

_VMEM_BYTES = 56 * 1024 * 1024

_LRN_ALPHA = 1e-4
_LRN_SIZE = 2
_LRN_K = 1.0


def _conv_body(xa_ref, xb_ref, w_ref, b_ref, o_ref, a_scr, *, shifts, cin,
               tr, relu, lrn):
    for t, s in enumerate(shifts):
        c0 = t * cin
        if s == 0:
            a_scr[:, c0:c0 + cin] = xa_ref[...]
        else:
            a_scr[0:tr - s, c0:c0 + cin] = xa_ref[s:tr, :]
            a_scr[tr - s:tr, c0:c0 + cin] = xb_ref[0:s, :]

    r = jnp.dot(a_scr[...], w_ref[...], preferred_element_type=jnp.float32)
    r = r + b_ref[...]
    if relu:
        r = jnp.maximum(r, 0.0)
    if lrn:
        xsq = r * r
        prev = pltpu.roll(xsq, 1, axis=1)
        lane = lax.broadcasted_iota(jnp.int32, r.shape, 1)
        prev = jnp.where(lane == 0, 0.0, prev)
        denom = _LRN_K + (_LRN_ALPHA / _LRN_SIZE) * (xsq + prev)
        inv = lax.rsqrt(denom)
        r = r * (inv * jnp.sqrt(inv))
    o_ref[...] = r.astype(o_ref.dtype)


def _pick_tr(rows, min_tr):
    for lo, hi in ((min_tr, 512), (512, 2048), (2048, rows)):
        for align in (8, 1):
            for t in range(hi, max(lo, min_tr), -1):
                if t % align == 0 and rows % t == 0:
                    return t
    raise ValueError("no valid row tile")


def _conv_flat(xf, w, b, *, wp, kh, kw, relu, lrn):
    rows, c = xf.shape
    cout = w.shape[1]
    shifts = tuple(i * wp + j for i in range(kh) for j in range(kw))
    assert w.shape[0] == len(shifts) * c
    tr = _pick_tr(rows, shifts[-1])
    nt = rows // tr
    bias = b.reshape(1, cout).astype(jnp.float32)

    return pl.pallas_call(
        functools.partial(_conv_body, shifts=shifts, cin=c, tr=tr,
                          relu=relu, lrn=lrn),
        out_shape=jax.ShapeDtypeStruct((rows, cout), jnp.bfloat16),
        grid=(nt,),
        in_specs=[
            pl.BlockSpec((tr, c), lambda i: (i, 0)),
            pl.BlockSpec((tr, c), lambda i: (jnp.minimum(i + 1, nt - 1), 0)),
            pl.BlockSpec(w.shape, lambda i: (0, 0)),
            pl.BlockSpec((1, cout), lambda i: (0, 0)),
        ],
        out_specs=pl.BlockSpec((tr, cout), lambda i: (i, 0)),
        scratch_shapes=[pltpu.VMEM((tr, len(shifts) * c), jnp.bfloat16)],
        compiler_params=pltpu.CompilerParams(
            dimension_semantics=("parallel",),
            vmem_limit_bytes=_VMEM_BYTES),
    )(xf, xf, w, bias)


def _pool_body(x_ref, s_ref, o_ref, m_scr, *, shifts, k):
    taps = [x_ref[0, s:s + k, :] for s in shifts]
    m_scr[...] = functools.reduce(jnp.maximum, taps)
    o_ref[0] = jnp.dot(s_ref[...], m_scr[...],
                       preferred_element_type=jnp.float32).astype(o_ref.dtype)


def _pool_selector(src_w, dst_h, dst_w, ring, po, k):
    sel = np.zeros((dst_h * dst_w, k), np.float32)
    for hi in range(dst_h):
        for wi in range(dst_w):
            ho, wo = hi - ring, wi - ring
            if 0 <= ho < po and 0 <= wo < po:
                sel[hi * dst_w + wi, 2 * ho * src_w + 2 * wo] = 1.0
    return jnp.asarray(sel, jnp.bfloat16)


def _maxpool(y, src_h, src_w, ho, *, dst_h, dst_w, ring, k):
    n, rows, c = y.shape
    po = (ho - 3) // 2 + 1
    shifts = tuple(i * src_w + j for i in range(3) for j in range(3))
    assert shifts[-1] + k <= rows
    sel = _pool_selector(src_w, dst_h, dst_w, ring, po, k)

    return pl.pallas_call(
        functools.partial(_pool_body, shifts=shifts, k=k),
        out_shape=jax.ShapeDtypeStruct((n, dst_h * dst_w, c), jnp.bfloat16),
        grid=(n,),
        in_specs=[
            pl.BlockSpec((1, rows, c), lambda i: (i, 0, 0)),
            pl.BlockSpec(sel.shape, lambda i: (0, 0)),
        ],
        out_specs=pl.BlockSpec((1, dst_h * dst_w, c), lambda i: (i, 0, 0)),
        scratch_shapes=[pltpu.VMEM((k, c), jnp.bfloat16)],
        compiler_params=pltpu.CompilerParams(
            dimension_semantics=("parallel",),
            vmem_limit_bytes=_VMEM_BYTES),
    )(y, sel)


def _conv1_group_input(x_nchw):
    xt = jnp.transpose(x_nchw, (0, 2, 3, 1))
    xt = jnp.pad(xt, ((0, 0), (0, 1), (0, 1), (0, 0)))
    pieces = [xt[:, ir::4, jr::4, :]
              for ir in range(4) for jr in range(4)]
    return jnp.concatenate(pieces, axis=-1).astype(jnp.bfloat16)


def _conv1_group_rows():
    rows = np.full((432,), 383, dtype=np.int32)
    g = 0
    for ig in range(3):
        for jg in range(3):
            for ir in range(4):
                for jr in range(4):
                    i, j = 4 * ig + ir, 4 * jg + jr
                    if i < 11 and j < 11:
                        for c in range(3):
                            rows[g * 48 + (ir * 4 + jr) * 3 + c] = \
                                (i * 11 + j) * 3 + c
            g += 1
    return rows


_CONV1_ROWS = _conv1_group_rows()


def _fc_body(a_ref, w_ref, b_ref, o_ref, *, relu):
    r = jnp.dot(a_ref[...], w_ref[...], preferred_element_type=jnp.float32)
    r = r + b_ref[...]
    if relu:
        r = jnp.maximum(r, 0.0)
    o_ref[...] = r.astype(o_ref.dtype)


def _fc(a, w, b, *, relu, tn, out_dtype):
    m, k = a.shape
    nn = w.shape[1]
    bias = b.reshape(1, nn).astype(jnp.float32)
    return pl.pallas_call(
        functools.partial(_fc_body, relu=relu),
        out_shape=jax.ShapeDtypeStruct((m, nn), out_dtype),
        grid=(nn // tn,),
        in_specs=[
            pl.BlockSpec((m, k), lambda j: (0, 0)),
            pl.BlockSpec((k, tn), lambda j: (0, j)),
            pl.BlockSpec((1, tn), lambda j: (0, j)),
        ],
        out_specs=pl.BlockSpec((m, tn), lambda j: (0, j)),
        compiler_params=pltpu.CompilerParams(
            dimension_semantics=("parallel",),
            vmem_limit_bytes=_VMEM_BYTES),
    )(a, w, bias)


def kernel(x, c1_w, c1_b, c2_w, c2_b, c3_w, c3_b, c4_w, c4_b, c5_w, c5_b,
           f1_w, f1_b, f2_w, f2_b, f3_w, f3_b):
    n = x.shape[0]

    x1 = _conv1_group_input(x)
    w1 = jnp.take(c1_w, jnp.asarray(_CONV1_ROWS), axis=0)
    y1 = _conv_flat(x1.reshape(n * 57 * 57, 48), w1, c1_b,
                    wp=57, kh=3, kw=3, relu=True, lrn=True)
    p1 = _maxpool(y1.reshape(n, 57 * 57, 128), 57, 57, 55,
                  dst_h=31, dst_w=31, ring=2, k=3024)

    y2 = _conv_flat(p1.reshape(n * 961, 128), c2_w, c2_b,
                    wp=31, kh=5, kw=5, relu=True, lrn=True)
    p2 = _maxpool(y2.reshape(n, 961, 256), 31, 31, 27,
                  dst_h=15, dst_w=15, ring=1, k=784)

    y3 = _conv_flat(p2.reshape(n * 225, 256), c3_w, c3_b,
                    wp=15, kh=3, kw=3, relu=True, lrn=False)
    x4 = jnp.pad(y3.reshape(n, 15, 15, 384)[:, :13, :13, :],
                 ((0, 0), (1, 1), (1, 1), (0, 0)))
    y4 = _conv_flat(x4.reshape(n * 225, 384), c4_w, c4_b,
                    wp=15, kh=3, kw=3, relu=True, lrn=False)
    x5 = jnp.pad(y4.reshape(n, 15, 15, 384)[:, :13, :13, :],
                 ((0, 0), (1, 1), (1, 1), (0, 0)))
    y5 = _conv_flat(x5.reshape(n * 225, 384), c5_w, c5_b,
                    wp=15, kh=3, kw=3, relu=True, lrn=False)
    p5 = _maxpool(y5.reshape(n, 225, 256), 15, 15, 13,
                  dst_h=6, dst_w=6, ring=0, k=176)

    f_in = jnp.transpose(p5, (0, 2, 1)).reshape(n, 256 * 6 * 6)
    h1 = _fc(f_in, f1_w, f1_b, relu=True, tn=512, out_dtype=jnp.bfloat16)
    h2 = _fc(h1, f2_w, f2_b, relu=True, tn=512, out_dtype=jnp.bfloat16)
    h3 = _fc(h2, f3_w, f3_b, relu=False, tn=512, out_dtype=jnp.float32)
    return h3[:, :1000]


# speedup vs baseline: 8.4970x; 1.0588x over previous
import functools

import numpy as np

import jax
import jax.numpy as jnp
from jax import lax
from jax.experimental import pallas as pl
from jax.experimental.pallas import tpu as pltpu

_VMEM_BYTES = 56 * 1024 * 1024

_LRN_ALPHA = 1e-4
_LRN_SIZE = 2
_LRN_K = 1.0


def _conv_body(xa_ref, xb_ref, w_ref, b_ref, o_ref, a_scr, *, shifts, cin,
               tr, relu, lrn):
    for t, s in enumerate(shifts):
        c0 = t * cin
        if s == 0:
            a_scr[:, c0:c0 + cin] = xa_ref[...]
        else:
            a_scr[0:tr - s, c0:c0 + cin] = xa_ref[s:tr, :]
            a_scr[tr - s:tr, c0:c0 + cin] = xb_ref[0:s, :]

    r = jnp.dot(a_scr[...], w_ref[...], preferred_element_type=jnp.float32)
    r = r + b_ref[...]
    if relu:
        r = jnp.maximum(r, 0.0)
    if lrn:
        xsq = r * r
        prev = pltpu.roll(xsq, 1, axis=1)
        lane = lax.broadcasted_iota(jnp.int32, r.shape, 1)
        prev = jnp.where(lane == 0, 0.0, prev)
        denom = _LRN_K + (_LRN_ALPHA / _LRN_SIZE) * (xsq + prev)
        inv = lax.rsqrt(denom)
        r = r * (inv * jnp.sqrt(inv))
    o_ref[...] = r.astype(o_ref.dtype)


def _pick_tr(rows, min_tr):
    for lo, hi in ((min_tr, 1280), (1280, 2048), (2048, rows)):
        for align in (8, 1):
            for t in range(hi, max(lo, min_tr), -1):
                if t % align == 0 and rows % t == 0:
                    return t
    raise ValueError("no valid row tile")


def _conv_flat(xf, w, b, *, wp, kh, kw, relu, lrn):
    rows, c = xf.shape
    cout = w.shape[1]
    shifts = tuple(i * wp + j for i in range(kh) for j in range(kw))
    assert w.shape[0] == len(shifts) * c
    tr = _pick_tr(rows, shifts[-1])
    nt = rows // tr
    bias = b.reshape(1, cout).astype(jnp.float32)

    return pl.pallas_call(
        functools.partial(_conv_body, shifts=shifts, cin=c, tr=tr,
                          relu=relu, lrn=lrn),
        out_shape=jax.ShapeDtypeStruct((rows, cout), jnp.bfloat16),
        grid=(nt,),
        in_specs=[
            pl.BlockSpec((tr, c), lambda i: (i, 0)),
            pl.BlockSpec((tr, c), lambda i: (jnp.minimum(i + 1, nt - 1), 0)),
            pl.BlockSpec(w.shape, lambda i: (0, 0)),
            pl.BlockSpec((1, cout), lambda i: (0, 0)),
        ],
        out_specs=pl.BlockSpec((tr, cout), lambda i: (i, 0)),
        scratch_shapes=[pltpu.VMEM((tr, len(shifts) * c), jnp.bfloat16)],
        compiler_params=pltpu.CompilerParams(
            dimension_semantics=("parallel",),
            vmem_limit_bytes=_VMEM_BYTES),
    )(xf, xf, w, bias)


def _pool_body(x_ref, s_ref, o_ref, m_scr, *, shifts, k):
    taps = [x_ref[0, s:s + k, :] for s in shifts]
    m_scr[...] = functools.reduce(jnp.maximum, taps)
    o_ref[0] = jnp.dot(s_ref[...], m_scr[...],
                       preferred_element_type=jnp.float32).astype(o_ref.dtype)


def _pool_selector(src_w, dst_h, dst_w, ring, po, k):
    sel = np.zeros((dst_h * dst_w, k), np.float32)
    for hi in range(dst_h):
        for wi in range(dst_w):
            ho, wo = hi - ring, wi - ring
            if 0 <= ho < po and 0 <= wo < po:
                sel[hi * dst_w + wi, 2 * ho * src_w + 2 * wo] = 1.0
    return jnp.asarray(sel, jnp.bfloat16)


def _maxpool(y, src_h, src_w, ho, *, dst_h, dst_w, ring, k):
    n, rows, c = y.shape
    po = (ho - 3) // 2 + 1
    shifts = tuple(i * src_w + j for i in range(3) for j in range(3))
    assert shifts[-1] + k <= rows
    sel = _pool_selector(src_w, dst_h, dst_w, ring, po, k)

    return pl.pallas_call(
        functools.partial(_pool_body, shifts=shifts, k=k),
        out_shape=jax.ShapeDtypeStruct((n, dst_h * dst_w, c), jnp.bfloat16),
        grid=(n,),
        in_specs=[
            pl.BlockSpec((1, rows, c), lambda i: (i, 0, 0)),
            pl.BlockSpec(sel.shape, lambda i: (0, 0)),
        ],
        out_specs=pl.BlockSpec((1, dst_h * dst_w, c), lambda i: (i, 0, 0)),
        scratch_shapes=[pltpu.VMEM((k, c), jnp.bfloat16)],
        compiler_params=pltpu.CompilerParams(
            dimension_semantics=("parallel",),
            vmem_limit_bytes=_VMEM_BYTES),
    )(y, sel)


def _conv1_group_input(x_nchw):
    xt = jnp.transpose(x_nchw, (0, 2, 3, 1))
    xt = jnp.pad(xt, ((0, 0), (0, 1), (0, 1), (0, 0)))
    pieces = [xt[:, ir::4, jr::4, :]
              for ir in range(4) for jr in range(4)]
    return jnp.concatenate(pieces, axis=-1).astype(jnp.bfloat16)


def _conv1_group_rows():
    rows = np.full((432,), 383, dtype=np.int32)
    g = 0
    for ig in range(3):
        for jg in range(3):
            for ir in range(4):
                for jr in range(4):
                    i, j = 4 * ig + ir, 4 * jg + jr
                    if i < 11 and j < 11:
                        for c in range(3):
                            rows[g * 48 + (ir * 4 + jr) * 3 + c] = \
                                (i * 11 + j) * 3 + c
            g += 1
    return rows


_CONV1_ROWS = _conv1_group_rows()


def _fc_body(a_ref, w_ref, b_ref, o_ref, *, relu):
    r = jnp.dot(a_ref[...], w_ref[...], preferred_element_type=jnp.float32)
    r = r + b_ref[...]
    if relu:
        r = jnp.maximum(r, 0.0)
    o_ref[...] = r.astype(o_ref.dtype)


def _fc(a, w, b, *, relu, tn, out_dtype):
    m, k = a.shape
    nn = w.shape[1]
    bias = b.reshape(1, nn).astype(jnp.float32)
    return pl.pallas_call(
        functools.partial(_fc_body, relu=relu),
        out_shape=jax.ShapeDtypeStruct((m, nn), out_dtype),
        grid=(nn // tn,),
        in_specs=[
            pl.BlockSpec((m, k), lambda j: (0, 0)),
            pl.BlockSpec((k, tn), lambda j: (0, j)),
            pl.BlockSpec((1, tn), lambda j: (0, j)),
        ],
        out_specs=pl.BlockSpec((m, tn), lambda j: (0, j)),
        compiler_params=pltpu.CompilerParams(
            dimension_semantics=("parallel",),
            vmem_limit_bytes=_VMEM_BYTES),
    )(a, w, bias)


def kernel(x, c1_w, c1_b, c2_w, c2_b, c3_w, c3_b, c4_w, c4_b, c5_w, c5_b,
           f1_w, f1_b, f2_w, f2_b, f3_w, f3_b):
    n = x.shape[0]

    x1 = _conv1_group_input(x)
    w1 = jnp.take(c1_w, jnp.asarray(_CONV1_ROWS), axis=0)
    y1 = _conv_flat(x1.reshape(n * 57 * 57, 48), w1, c1_b,
                    wp=57, kh=3, kw=3, relu=True, lrn=True)
    p1 = _maxpool(y1.reshape(n, 57 * 57, 128), 57, 57, 55,
                  dst_h=31, dst_w=31, ring=2, k=3024)

    y2 = _conv_flat(p1.reshape(n * 961, 128), c2_w, c2_b,
                    wp=31, kh=5, kw=5, relu=True, lrn=True)
    p2 = _maxpool(y2.reshape(n, 961, 256), 31, 31, 27,
                  dst_h=15, dst_w=15, ring=1, k=784)

    y3 = _conv_flat(p2.reshape(n * 225, 256), c3_w, c3_b,
                    wp=15, kh=3, kw=3, relu=True, lrn=False)
    x4 = jnp.pad(y3.reshape(n, 15, 15, 384)[:, :13, :13, :],
                 ((0, 0), (1, 1), (1, 1), (0, 0)))
    y4 = _conv_flat(x4.reshape(n * 225, 384), c4_w, c4_b,
                    wp=15, kh=3, kw=3, relu=True, lrn=False)
    x5 = jnp.pad(y4.reshape(n, 15, 15, 384)[:, :13, :13, :],
                 ((0, 0), (1, 1), (1, 1), (0, 0)))
    y5 = _conv_flat(x5.reshape(n * 225, 384), c5_w, c5_b,
                    wp=15, kh=3, kw=3, relu=True, lrn=False)
    p5 = _maxpool(y5.reshape(n, 225, 256), 15, 15, 13,
                  dst_h=6, dst_w=6, ring=0, k=176)

    f_in = jnp.transpose(p5, (0, 2, 1)).reshape(n, 256 * 6 * 6)
    h1 = _fc(f_in, f1_w, f1_b, relu=True, tn=512, out_dtype=jnp.bfloat16)
    h2 = _fc(h1, f2_w, f2_b, relu=True, tn=512, out_dtype=jnp.bfloat16)
    h3 = _fc(h2, f3_w, f3_b, relu=False, tn=512, out_dtype=jnp.float32)
    return h3[:, :1000]


# speedup vs baseline: 10.3646x; 1.2198x over previous
import functools

import numpy as np

import jax
import jax.numpy as jnp
from jax import lax
from jax.experimental import pallas as pl
from jax.experimental.pallas import tpu as pltpu

_VMEM_BYTES = 56 * 1024 * 1024

_LRN_ALPHA = 1e-4
_LRN_SIZE = 2
_LRN_K = 1.0


def _conv_body(xa_ref, xb_ref, w_ref, b_ref, o_ref, a_scr, *, shifts, cin,
               tr, relu, lrn):
    for t, s in enumerate(shifts):
        c0 = t * cin
        if s == 0:
            a_scr[:, c0:c0 + cin] = xa_ref[...]
        else:
            a_scr[0:tr - s, c0:c0 + cin] = xa_ref[s:tr, :]
            a_scr[tr - s:tr, c0:c0 + cin] = xb_ref[0:s, :]

    r = jnp.dot(a_scr[...], w_ref[...], preferred_element_type=jnp.float32)
    r = r + b_ref[...]
    if relu:
        r = jnp.maximum(r, 0.0)
    if lrn:
        xsq = r * r
        prev = pltpu.roll(xsq, 1, axis=1)
        lane = lax.broadcasted_iota(jnp.int32, r.shape, 1)
        prev = jnp.where(lane == 0, 0.0, prev)
        denom = _LRN_K + (_LRN_ALPHA / _LRN_SIZE) * (xsq + prev)
        inv = lax.rsqrt(denom)
        r = r * (inv * jnp.sqrt(inv))
    o_ref[...] = r.astype(o_ref.dtype)


def _pick_tr(rows, min_tr):
    for lo, hi in ((min_tr, 1280), (1280, 2048), (2048, rows)):
        for align in (8, 1):
            for t in range(hi, max(lo, min_tr), -1):
                if t % align == 0 and rows % t == 0:
                    return t
    raise ValueError("no valid row tile")


def _conv_flat(xf, w, b, *, wp, kh, kw, relu, lrn):
    rows, c = xf.shape
    cout = w.shape[1]
    shifts = tuple(i * wp + j for i in range(kh) for j in range(kw))
    assert w.shape[0] == len(shifts) * c
    tr = _pick_tr(rows, shifts[-1])
    nt = rows // tr
    bias = b.reshape(1, cout).astype(jnp.float32)

    return pl.pallas_call(
        functools.partial(_conv_body, shifts=shifts, cin=c, tr=tr,
                          relu=relu, lrn=lrn),
        out_shape=jax.ShapeDtypeStruct((rows, cout), jnp.bfloat16),
        grid=(nt,),
        in_specs=[
            pl.BlockSpec((tr, c), lambda i: (i, 0)),
            pl.BlockSpec((tr, c), lambda i: (jnp.minimum(i + 1, nt - 1), 0)),
            pl.BlockSpec(w.shape, lambda i: (0, 0)),
            pl.BlockSpec((1, cout), lambda i: (0, 0)),
        ],
        out_specs=pl.BlockSpec((tr, cout), lambda i: (i, 0)),
        scratch_shapes=[pltpu.VMEM((tr, len(shifts) * c), jnp.bfloat16)],
        compiler_params=pltpu.CompilerParams(
            dimension_semantics=("parallel",),
            vmem_limit_bytes=_VMEM_BYTES),
    )(xf, xf, w, bias)


def _pool_body(x_ref, s_ref, o_ref, m_scr, *, shifts, k):
    taps = [x_ref[0, s:s + k, :] for s in shifts]
    m_scr[...] = functools.reduce(jnp.maximum, taps)
    o_ref[0] = jnp.dot(s_ref[...], m_scr[...],
                       preferred_element_type=jnp.float32).astype(o_ref.dtype)


def _pool_selector(src_w, dst_h, dst_w, ring, po, k):
    sel = np.zeros((dst_h * dst_w, k), np.float32)
    for hi in range(dst_h):
        for wi in range(dst_w):
            ho, wo = hi - ring, wi - ring
            if 0 <= ho < po and 0 <= wo < po:
                sel[hi * dst_w + wi, 2 * ho * src_w + 2 * wo] = 1.0
    return jnp.asarray(sel, jnp.bfloat16)


def _maxpool(y, src_h, src_w, ho, *, dst_h, dst_w, ring, k):
    n, rows, c = y.shape
    po = (ho - 3) // 2 + 1
    shifts = tuple(i * src_w + j for i in range(3) for j in range(3))
    assert shifts[-1] + k <= rows
    sel = _pool_selector(src_w, dst_h, dst_w, ring, po, k)

    return pl.pallas_call(
        functools.partial(_pool_body, shifts=shifts, k=k),
        out_shape=jax.ShapeDtypeStruct((n, dst_h * dst_w, c), jnp.bfloat16),
        grid=(n,),
        in_specs=[
            pl.BlockSpec((1, rows, c), lambda i: (i, 0, 0)),
            pl.BlockSpec(sel.shape, lambda i: (0, 0)),
        ],
        out_specs=pl.BlockSpec((1, dst_h * dst_w, c), lambda i: (i, 0, 0)),
        scratch_shapes=[pltpu.VMEM((k, c), jnp.bfloat16)],
        compiler_params=pltpu.CompilerParams(
            dimension_semantics=("parallel",),
            vmem_limit_bytes=_VMEM_BYTES),
    )(y, sel)


def _conv1_group_input(x_nchw):
    xt = jnp.transpose(x_nchw.astype(jnp.bfloat16), (0, 2, 3, 1))
    xt = jnp.pad(xt, ((0, 0), (0, 1), (0, 1), (0, 0)))
    pieces = [xt[:, ir::4, jr::4, :]
              for ir in range(4) for jr in range(4)]
    return jnp.concatenate(pieces, axis=-1)


def _conv1_group_rows():
    rows = np.full((432,), 383, dtype=np.int32)
    g = 0
    for ig in range(3):
        for jg in range(3):
            for ir in range(4):
                for jr in range(4):
                    i, j = 4 * ig + ir, 4 * jg + jr
                    if i < 11 and j < 11:
                        for c in range(3):
                            rows[g * 48 + (ir * 4 + jr) * 3 + c] = \
                                (i * 11 + j) * 3 + c
            g += 1
    return rows


_CONV1_ROWS = _conv1_group_rows()


def _fc_body(a_ref, w_ref, b_ref, o_ref, *, relu):
    r = jnp.dot(a_ref[...], w_ref[...], preferred_element_type=jnp.float32)
    r = r + b_ref[...]
    if relu:
        r = jnp.maximum(r, 0.0)
    o_ref[...] = r.astype(o_ref.dtype)


def _fc(a, w, b, *, relu, tn, out_dtype):
    m, k = a.shape
    nn = w.shape[1]
    bias = b.reshape(1, nn).astype(jnp.float32)
    return pl.pallas_call(
        functools.partial(_fc_body, relu=relu),
        out_shape=jax.ShapeDtypeStruct((m, nn), out_dtype),
        grid=(nn // tn,),
        in_specs=[
            pl.BlockSpec((m, k), lambda j: (0, 0)),
            pl.BlockSpec((k, tn), lambda j: (0, j)),
            pl.BlockSpec((1, tn), lambda j: (0, j)),
        ],
        out_specs=pl.BlockSpec((m, tn), lambda j: (0, j)),
        compiler_params=pltpu.CompilerParams(
            dimension_semantics=("parallel",),
            vmem_limit_bytes=_VMEM_BYTES),
    )(a, w, bias)


def kernel(x, c1_w, c1_b, c2_w, c2_b, c3_w, c3_b, c4_w, c4_b, c5_w, c5_b,
           f1_w, f1_b, f2_w, f2_b, f3_w, f3_b):
    n = x.shape[0]

    x1 = _conv1_group_input(x)
    w1 = jnp.take(c1_w, jnp.asarray(_CONV1_ROWS), axis=0)
    y1 = _conv_flat(x1.reshape(n * 57 * 57, 48), w1, c1_b,
                    wp=57, kh=3, kw=3, relu=True, lrn=True)
    p1 = _maxpool(y1.reshape(n, 57 * 57, 128), 57, 57, 55,
                  dst_h=31, dst_w=31, ring=2, k=3024)

    y2 = _conv_flat(p1.reshape(n * 961, 128), c2_w, c2_b,
                    wp=31, kh=5, kw=5, relu=True, lrn=True)
    p2 = _maxpool(y2.reshape(n, 961, 256), 31, 31, 27,
                  dst_h=15, dst_w=15, ring=1, k=784)

    y3 = _conv_flat(p2.reshape(n * 225, 256), c3_w, c3_b,
                    wp=15, kh=3, kw=3, relu=True, lrn=False)
    x4 = jnp.pad(y3.reshape(n, 15, 15, 384)[:, :13, :13, :],
                 ((0, 0), (1, 1), (1, 1), (0, 0)))
    y4 = _conv_flat(x4.reshape(n * 225, 384), c4_w, c4_b,
                    wp=15, kh=3, kw=3, relu=True, lrn=False)
    x5 = jnp.pad(y4.reshape(n, 15, 15, 384)[:, :13, :13, :],
                 ((0, 0), (1, 1), (1, 1), (0, 0)))
    y5 = _conv_flat(x5.reshape(n * 225, 384), c5_w, c5_b,
                    wp=15, kh=3, kw=3, relu=True, lrn=False)
    p5 = _maxpool(y5.reshape(n, 225, 256), 15, 15, 13,
                  dst_h=6, dst_w=6, ring=0, k=176)

    f_in = jnp.transpose(p5, (0, 2, 1)).reshape(n, 256 * 6 * 6)
    h1 = _fc(f_in, f1_w, f1_b, relu=True, tn=512, out_dtype=jnp.bfloat16)
    h2 = _fc(h1, f2_w, f2_b, relu=True, tn=512, out_dtype=jnp.bfloat16)
    h3 = _fc(h2, f3_w, f3_b, relu=False, tn=512, out_dtype=jnp.float32)
    return h3[:, :1000]
